# baseline (speedup 1.0000x reference)
import functools

import jax
import jax.numpy as jnp
from jax import lax
from jax.experimental import pallas as pl
from jax.experimental.pallas import tpu as pltpu
from jax.experimental.pallas import tpu_sc as plsc

_NC = 2
_NS = 16
_NW = _NC * _NS

_D = 32
_R = 16384
_S = 50
_L = 16
_SPW = _R // 128 // _NW
_T = _SPW * _S
_NB = 4


def _lookup_body(wp_hbm, xt_hbm, g_hbm, idx_v, i4s, rs, os_, gss, oss):
    wid = lax.axis_index("s") * _NC + lax.axis_index("c")
    ih0 = wid * _SPW

    pltpu.sync_copy(xt_hbm.at[:, pl.ds(ih0 * 128, _SPW * 128)], idx_v)

    def _sl_j(t):
        sl = lax.div(t, _S)
        return sl, t - sl * _S

    def _fill_idx4(t, i4):
        sl, j = _sl_j(t)
        c0 = sl * 128
        for g in range(8):
            v = idx_v[j, pl.ds(c0 + g * _L, _L)]
            i4[pl.ds(g * _L, _L)] = lax.shift_right_logical(v, 2)

    def _start_gather(t, i4, rows, sem):
        @pl.when(t < _T)
        def _():
            _fill_idx4(t, i4)
            pltpu.async_copy(wp_hbm.at[i4], rows, sem)

    def _step(t, i4, rows, out, gsem, osem, first):
        sl, j = _sl_j(t)
        c0 = sl * 128
        pltpu.make_async_copy(wp_hbm.at[i4], rows, gsem).wait()

        @pl.when(jnp.logical_not(first))
        def _():
            for kh in range(4):
                pltpu.make_async_copy(g_hbm.at[0, kh, 0], out.at[kh], osem).wait()

        bases = []
        rowis = []
        for g in range(8):
            v = idx_v[j, pl.ds(c0 + g * _L, _L)]
            bases.append(lax.mul(lax.bitwise_and(v, 3), _D))
            rowis.append(lax.iota(jnp.int32, _L) + g * _L)
        for k in range(_D):
            vals = [plsc.load_gather(rows, [rowis[g], bases[g] + k])
                    for g in range(8)]
            for g in range(8):
                out[k // 8, k % 8, pl.ds(g * _L, _L)] = vals[g]
        return sl, j

    def _fire_out(sl, j, out, osem):
        for kh in range(4):
            pltpu.async_copy(out.at[kh], g_hbm.at[j, kh, ih0 + sl], osem)

    for b in range(_NB):
        _start_gather(b, i4s[b], rs[b], gss[b])

    def quad(m, carry):
        t0 = m * _NB
        for b in range(_NB):
            sl, j = _step(t0 + b, i4s[b], rs[b], os_[b], gss[b], oss[b],
                          m == 0)
            _start_gather(t0 + b + _NB, i4s[b], rs[b], gss[b])
            _fire_out(sl, j, os_[b], oss[b])
        return carry

    lax.fori_loop(0, _T // _NB, quad, 0)
    for b in range(_NB):
        for kh in range(4):
            pltpu.make_async_copy(g_hbm.at[0, kh, 0], os_[b].at[kh],
                                  oss[b]).wait()


_V = 1000000
_FULL = _V // 128
_TAIL = _V - _FULL * 128


def _pack_body(wt_hbm, wp_hbm, in_v, out_v, tin_v, tout_v):
    wid = lax.axis_index("s") * _NC + lax.axis_index("c")
    nslabs = jnp.where(wid < _FULL % _NW, _FULL // _NW + 1, _FULL // _NW)

    def slab(s, carry):
        c0 = pl.multiple_of((s * _NW + wid) * 128, 128)
        pltpu.sync_copy(wt_hbm.at[:, pl.ds(c0, 128)], in_v)
        for c in range(128):
            cc = jnp.full((_L,), c, jnp.int32)
            for h in range(2):
                rowk = lax.iota(jnp.int32, _L) + h * _L
                out_v[c // 4, pl.ds((c % 4) * _D + h * _L, _L)] = (
                    plsc.load_gather(in_v, [rowk, cc]))
        pltpu.sync_copy(out_v, wp_hbm.at[pl.ds(pl.multiple_of(c0 // 4, 32), 32), :])
        return carry

    lax.fori_loop(0, nslabs, slab, 0)

    @pl.when(wid == _NW - 1)
    def _():
        c0 = _FULL * 128
        pltpu.sync_copy(wt_hbm.at[:, pl.ds(c0, _TAIL)], tin_v)
        for c in range(_TAIL):
            cc = jnp.full((_L,), c, jnp.int32)
            for h in range(2):
                rowk = lax.iota(jnp.int32, _L) + h * _L
                tout_v[c // 4, pl.ds((c % 4) * _D + h * _L, _L)] = (
                    plsc.load_gather(tin_v, [rowk, cc]))
        pltpu.sync_copy(tout_v, wp_hbm.at[pl.ds(c0 // 4, _TAIL // 4), :])


@jax.jit
def _pack_table(Wt):
    mesh = plsc.VectorSubcoreMesh(core_axis_name="c", subcore_axis_name="s")
    f = functools.partial(
        pl.kernel,
        mesh=mesh,
        out_type=jax.ShapeDtypeStruct((_D * _V // 128, 128), jnp.float32),
        scratch_types=[
            pltpu.VMEM((_D, 128), jnp.float32),
            pltpu.VMEM((32, 128), jnp.float32),
            pltpu.VMEM((_D, _TAIL), jnp.float32),
            pltpu.VMEM((_TAIL // 4, 128), jnp.float32),
        ],
        compiler_params=pltpu.CompilerParams(needs_layout_passes=False),
    )(_pack_body)
    return f(Wt)


@jax.jit
def _embedding_lookup(Wp, xt):
    mesh = plsc.VectorSubcoreMesh(core_axis_name="c", subcore_axis_name="s")
    f = functools.partial(
        pl.kernel,
        mesh=mesh,
        out_type=jax.ShapeDtypeStruct((_S, _D // 8, _R // 128, 8, 128),
                                      jnp.float32),
        scratch_types=[
            pltpu.VMEM((_S, _SPW * 128), jnp.int32),
            [pltpu.VMEM((128,), jnp.int32) for _ in range(_NB)],
            [pltpu.VMEM((128, 128), jnp.float32) for _ in range(_NB)],
            [pltpu.VMEM((_D // 8, 8, 128), jnp.float32) for _ in range(_NB)],
            [pltpu.SemaphoreType.DMA for _ in range(_NB)],
            [pltpu.SemaphoreType.DMA for _ in range(_NB)],
        ],
        compiler_params=pltpu.CompilerParams(needs_layout_passes=False),
    )(_lookup_body)
    return f(Wp, xt)


def kernel(x, W):
    Wp = _pack_table(W.T)
    G = _embedding_lookup(Wp, x.T)
    return G.transpose(2, 4, 0, 1, 3).reshape(_R, _S, _D)


# speedup vs baseline: 1.5700x; 1.5700x over previous
import functools

import jax
import jax.numpy as jnp
from jax import lax
from jax.experimental import pallas as pl
from jax.experimental.pallas import tpu as pltpu
from jax.experimental.pallas import tpu_sc as plsc

_NC = 2
_NS = 16
_NW = _NC * _NS

_D = 32
_R = 16384
_S = 50
_L = 16
_SPW = _R // 128 // _NW
_T = _SPW * _S
_NB = 4


def _lookup_body(wp_hbm, xt_hbm, g_hbm, idx_v, i4s, rs, os_, gss, oss):
    wid = lax.axis_index("s") * _NC + lax.axis_index("c")
    ih0 = wid * _SPW

    pltpu.sync_copy(xt_hbm.at[:, pl.ds(ih0 * 128, _SPW * 128)], idx_v)

    def _sl_j(t):
        sl = lax.div(t, _S)
        return sl, t - sl * _S

    def _fill_idx4(t, i4):
        sl, j = _sl_j(t)
        c0 = sl * 128
        for g in range(8):
            v = idx_v[j, pl.ds(c0 + g * _L, _L)]
            i4[pl.ds(g * _L, _L)] = lax.shift_right_logical(v, 2)

    def _start_gather(t, i4, rows, sem):
        @pl.when(t < _T)
        def _():
            _fill_idx4(t, i4)
            pltpu.async_copy(wp_hbm.at[i4], rows, sem)

    def _step(t, i4, rows, out, gsem, osem, first):
        sl, j = _sl_j(t)
        c0 = sl * 128
        pltpu.make_async_copy(wp_hbm.at[i4], rows, gsem).wait()

        @pl.when(jnp.logical_not(first))
        def _():
            for kh in range(4):
                pltpu.make_async_copy(g_hbm.at[0, kh, 0], out.at[kh], osem).wait()

        bases = []
        rowis = []
        for g in range(8):
            v = idx_v[j, pl.ds(c0 + g * _L, _L)]
            bases.append(lax.mul(lax.bitwise_and(v, 3), _D))
            rowis.append(lax.iota(jnp.int32, _L) + g * _L)
        for k in range(_D):
            vals = [plsc.load_gather(rows, [rowis[g], bases[g] + k])
                    for g in range(8)]
            for g in range(8):
                out[k // 8, k % 8, pl.ds(g * _L, _L)] = vals[g]
        return sl, j

    def _fire_out(sl, j, out, osem):
        for kh in range(4):
            pltpu.async_copy(out.at[kh], g_hbm.at[j, kh, ih0 + sl], osem)

    for b in range(_NB):
        _start_gather(b, i4s[b], rs[b], gss[b])

    def quad(m, carry):
        t0 = m * _NB
        for b in range(_NB):
            sl, j = _step(t0 + b, i4s[b], rs[b], os_[b], gss[b], oss[b],
                          m == 0)
            _start_gather(t0 + b + _NB, i4s[b], rs[b], gss[b])
            _fire_out(sl, j, os_[b], oss[b])
        return carry

    lax.fori_loop(0, _T // _NB, quad, 0)
    for b in range(_NB):
        for kh in range(4):
            pltpu.make_async_copy(g_hbm.at[0, kh, 0], os_[b].at[kh],
                                  oss[b]).wait()


@jax.jit
def _embedding_lookup(Wp, xt):
    mesh = plsc.VectorSubcoreMesh(core_axis_name="c", subcore_axis_name="s")
    f = functools.partial(
        pl.kernel,
        mesh=mesh,
        out_type=jax.ShapeDtypeStruct((_S, _D // 8, _R // 128, 8, 128),
                                      jnp.float32),
        scratch_types=[
            pltpu.VMEM((_S, _SPW * 128), jnp.int32),
            [pltpu.VMEM((128,), jnp.int32) for _ in range(_NB)],
            [pltpu.VMEM((128, 128), jnp.float32) for _ in range(_NB)],
            [pltpu.VMEM((_D // 8, 8, 128), jnp.float32) for _ in range(_NB)],
            [pltpu.SemaphoreType.DMA for _ in range(_NB)],
            [pltpu.SemaphoreType.DMA for _ in range(_NB)],
        ],
        compiler_params=pltpu.CompilerParams(needs_layout_passes=False),
    )(_lookup_body)
    return f(Wp, xt)


def kernel(x, W):
    Wp = W.reshape(_D * 1000000 // 128, 128)
    G = _embedding_lookup(Wp, x.T)
    return G.transpose(2, 4, 0, 1, 3).reshape(_R, _S, _D)
